# baseline (speedup 1.0000x reference)
import functools

import jax
import jax.numpy as jnp
from jax import lax
from jax.experimental import pallas as pl
from jax.experimental.pallas import tpu as pltpu
from jax.experimental.pallas import tpu_sc as plsc

B = 16384
D = 128
NNODES = 1_000_000
NS = 16
L = 16
RANGE = 62504
CHUNK = B // NS
SUB = 128
NSUB = CHUNK // SUB
P1C = 1024
NP1 = B // P1C


def _body(nodes_hbm, values_hbm, ts_hbm, mem_out_hbm, lu_out_hbm,
          nodes_a, nodes_b, win_v, j_all, lu_all, rows_a, rows_b, fix_v,
          sem_n, sem_j, sem_ts, sem_ga, sem_gb, sem_wa, sem_wb,
          win_hbm):
    sid = lax.axis_index("s")
    base = sid * RANGE
    my = sid * CHUNK
    iota = lax.broadcasted_iota(jnp.int32, (L,), 0)
    nxt_idx = jnp.minimum(iota + 1, L - 1)
    notlast = iota < L - 1
    nbufs = (nodes_a, nodes_b)

    rbufs = (rows_a, rows_b)
    gsems = (sem_ga, sem_gb)
    wsems = (sem_wa, sem_wb)
    gcp = [None, None]
    wcp = [None, None]
    gcp[0] = pltpu.async_copy(values_hbm.at[pl.ds(my, SUB)], rows_a, sem_ga)

    ncp = pltpu.async_copy(nodes_hbm.at[pl.ds(0, P1C)], nodes_a, sem_n)

    for k in range(NP1):
        ncp.wait()
        if k + 1 < NP1:
            ncp = pltpu.async_copy(nodes_hbm.at[pl.ds((k + 1) * P1C, P1C)],
                                   nbufs[(k + 1) % 2], sem_n)
        nbuf = nbufs[k % 2]

        def p1(t, carry, _k=k, _nbuf=nbuf):
            n = _nbuf[pl.ds(t * L, L)]
            rel = n - base
            m = (rel >= 0) & (rel < RANGE)
            jv = _k * P1C + t * L + iota
            plsc.store_scatter(win_v, [rel], jv, mask=m)
            r = plsc.load_gather(win_v, [rel], mask=m)
            bad = m & (r != jv)

            @pl.when(jnp.any(bad))
            def _():
                key = n * L + iota
                skey, _ = plsc.sort_key_val(key, key)
                n_s = skey >> 4
                j_s = _k * P1C + t * L + (skey & (L - 1))
                nxt = n_s.at[nxt_idx].get(mode="promise_in_bounds")
                loser = (n_s == nxt) & notlast
                rel_s = n_s - base
                m2 = ((rel_s >= 0) & (rel_s < RANGE)
                      & jnp.logical_not(loser))
                plsc.store_scatter(win_v, [rel_s], j_s, mask=m2)

            return carry

        lax.fori_loop(0, P1C // L, p1, 0)

        if k % 2 == 0:
            s = k // 2
            b = s % 2
            gcp[b].wait()
            wcp[b] = pltpu.async_copy(
                rbufs[b], mem_out_hbm.at[pl.ds(my + s * SUB, SUB)], wsems[b])
            if s + 1 < NSUB:
                nb = (s + 1) % 2
                if s >= 1:
                    wcp[nb].wait()
                gcp[nb] = pltpu.async_copy(
                    values_hbm.at[pl.ds(my + (s + 1) * SUB, SUB)],
                    rbufs[nb], gsems[nb])

    wcp[0].wait()
    wcp[1].wait()

    pltpu.sync_copy(win_v, win_hbm.at[pl.ds(base, RANGE)])
    plsc.subcore_barrier()

    pltpu.sync_copy(nodes_hbm.at[pl.ds(my, CHUNK)], nodes_a)
    jcps = [pltpu.async_copy(win_hbm.at[nodes_a.at[pl.ds(c * SUB, SUB)]],
                             j_all.at[pl.ds(c * SUB, SUB)], sem_j)
            for c in range(NSUB)]
    for c in jcps:
        c.wait()

    tcps = [pltpu.async_copy(ts_hbm.at[j_all.at[pl.ds(c * SUB, SUB)]],
                             lu_all.at[pl.ds(c * SUB, SUB)], sem_ts)
            for c in range(NSUB)]

    def fix(g, carry):
        jv = j_all[pl.ds(g * L, L)]
        expect = my + g * L + iota
        bad = jv != expect

        @pl.when(jnp.any(bad))
        def _():
            pltpu.sync_copy(values_hbm.at[j_all.at[pl.ds(g * L, L)]], fix_v)
            pltpu.sync_copy(fix_v, mem_out_hbm.at[pl.ds(my + g * L, L)])

        return carry

    lax.fori_loop(0, CHUNK // L, fix, 0)

    for c in tcps:
        c.wait()
    pltpu.sync_copy(lu_all, lu_out_hbm.at[pl.ds(my, CHUNK)])


_dedup_gather = functools.partial(
    pl.kernel,
    out_type=(
        jax.ShapeDtypeStruct((B, D), jnp.float32),
        jax.ShapeDtypeStruct((B,), jnp.float32),
    ),
    mesh=plsc.VectorSubcoreMesh(core_axis_name="c", subcore_axis_name="s",
                                num_cores=1),
    compiler_params=pltpu.CompilerParams(needs_layout_passes=False),
    scratch_types=[
        pltpu.VMEM((P1C,), jnp.int32),
        pltpu.VMEM((P1C,), jnp.int32),
        pltpu.VMEM((RANGE,), jnp.int32),
        pltpu.VMEM((CHUNK,), jnp.int32),
        pltpu.VMEM((CHUNK,), jnp.float32),
        pltpu.VMEM((SUB, D), jnp.float32),
        pltpu.VMEM((SUB, D), jnp.float32),
        pltpu.VMEM((L, D), jnp.float32),
        pltpu.SemaphoreType.DMA,
        pltpu.SemaphoreType.DMA,
        pltpu.SemaphoreType.DMA,
        pltpu.SemaphoreType.DMA,
        pltpu.SemaphoreType.DMA,
        pltpu.SemaphoreType.DMA,
        pltpu.SemaphoreType.DMA,
        pltpu.HBM((NS * RANGE,), jnp.int32),
    ],
)(_body)


def kernel(memory, last_update, nodes, values, ts):
    mem_out, lu_out = _dedup_gather(nodes, values, ts)
    return mem_out, lu_out


# speedup vs baseline: 1.5437x; 1.5437x over previous
import functools

import jax
import jax.numpy as jnp
from jax import lax
from jax.experimental import pallas as pl
from jax.experimental.pallas import tpu as pltpu
from jax.experimental.pallas import tpu_sc as plsc

B = 16384
D = 128
NNODES = 1_000_000
NS = 16
L = 16
RANGE = 62504
CHUNK = B // NS
SUB = 128
NSUB = CHUNK // SUB
P1C = 1024
NP1 = B // P1C


def _body(nodes_hbm, values_hbm, ts_hbm, mem_out_hbm, lu_out_hbm,
          nodes_v, win_v, j_all, lu_all, rows_a, rows_b, fix_v,
          sem_j, sem_ts, sem_ga, sem_gb, sem_wa, sem_wb,
          win_hbm):
    sid = lax.axis_index("s")
    base = sid * RANGE
    my = sid * CHUNK
    iota = lax.broadcasted_iota(jnp.int32, (L,), 0)
    nxt_idx = jnp.minimum(iota + 1, L - 1)
    notlast = iota < L - 1

    rbufs = (rows_a, rows_b)
    gsems = (sem_ga, sem_gb)
    wsems = (sem_wa, sem_wb)
    gcp = [None, None]
    wcp = [None, None]
    gcp[0] = pltpu.async_copy(values_hbm.at[pl.ds(my, SUB)], rows_a, sem_ga)

    pltpu.sync_copy(nodes_hbm, nodes_v)

    for k in range(NP1):
        koff = k * P1C

        def p1(t, acc, _koff=koff):
            n = nodes_v[pl.ds(_koff + t * L, L)]
            rel = n - base
            m = (rel >= 0) & (rel < RANGE)
            jv = _koff + t * L + iota
            plsc.store_scatter(win_v, [rel], jv, mask=m)
            r = plsc.load_gather(win_v, [rel], mask=m)
            bad = m & (r != jv)
            return acc + plsc.all_reduce_population_count(bad)

        acc = lax.fori_loop(0, P1C // L, p1, jnp.zeros((L,), jnp.int32))

        @pl.when(jnp.any(acc > 0))
        def _(_koff=koff):
            def rp(t, carry):
                n = nodes_v[pl.ds(_koff + t * L, L)]
                key = n * L + iota
                skey, _ = plsc.sort_key_val(key, key)
                n_s = skey >> 4
                j_s = _koff + t * L + (skey & (L - 1))
                nxt = n_s.at[nxt_idx].get(mode="promise_in_bounds")
                loser = (n_s == nxt) & notlast
                rel_s = n_s - base
                m2 = ((rel_s >= 0) & (rel_s < RANGE)
                      & jnp.logical_not(loser))
                plsc.store_scatter(win_v, [rel_s], j_s, mask=m2)
                return carry

            lax.fori_loop(0, P1C // L, rp, 0)

        if k % 2 == 0:
            s = k // 2
            b = s % 2
            gcp[b].wait()
            wcp[b] = pltpu.async_copy(
                rbufs[b], mem_out_hbm.at[pl.ds(my + s * SUB, SUB)], wsems[b])
            if s + 1 < NSUB:
                nb = (s + 1) % 2
                if s >= 1:
                    wcp[nb].wait()
                gcp[nb] = pltpu.async_copy(
                    values_hbm.at[pl.ds(my + (s + 1) * SUB, SUB)],
                    rbufs[nb], gsems[nb])

    wcp[0].wait()
    wcp[1].wait()

    pltpu.sync_copy(win_v, win_hbm.at[pl.ds(base, RANGE)])
    plsc.subcore_barrier()

    jcps = [pltpu.async_copy(win_hbm.at[nodes_v.at[pl.ds(my + c * SUB, SUB)]],
                             j_all.at[pl.ds(c * SUB, SUB)], sem_j)
            for c in range(NSUB)]
    for c in jcps:
        c.wait()

    tcps = [pltpu.async_copy(ts_hbm.at[j_all.at[pl.ds(c * SUB, SUB)]],
                             lu_all.at[pl.ds(c * SUB, SUB)], sem_ts)
            for c in range(NSUB)]

    def fix(g, carry):
        jv = j_all[pl.ds(g * L, L)]
        expect = my + g * L + iota
        bad = jv != expect

        @pl.when(jnp.any(bad))
        def _():
            pltpu.sync_copy(values_hbm.at[j_all.at[pl.ds(g * L, L)]], fix_v)
            pltpu.sync_copy(fix_v, mem_out_hbm.at[pl.ds(my + g * L, L)])

        return carry

    lax.fori_loop(0, CHUNK // L, fix, 0)

    for c in tcps:
        c.wait()
    pltpu.sync_copy(lu_all, lu_out_hbm.at[pl.ds(my, CHUNK)])


_dedup_gather = functools.partial(
    pl.kernel,
    out_type=(
        jax.ShapeDtypeStruct((B, D), jnp.float32),
        jax.ShapeDtypeStruct((B,), jnp.float32),
    ),
    mesh=plsc.VectorSubcoreMesh(core_axis_name="c", subcore_axis_name="s",
                                num_cores=1),
    compiler_params=pltpu.CompilerParams(needs_layout_passes=False),
    scratch_types=[
        pltpu.VMEM((B,), jnp.int32),
        pltpu.VMEM((RANGE,), jnp.int32),
        pltpu.VMEM((CHUNK,), jnp.int32),
        pltpu.VMEM((CHUNK,), jnp.float32),
        pltpu.VMEM((SUB, D), jnp.float32),
        pltpu.VMEM((SUB, D), jnp.float32),
        pltpu.VMEM((L, D), jnp.float32),
        pltpu.SemaphoreType.DMA,
        pltpu.SemaphoreType.DMA,
        pltpu.SemaphoreType.DMA,
        pltpu.SemaphoreType.DMA,
        pltpu.SemaphoreType.DMA,
        pltpu.SemaphoreType.DMA,
        pltpu.HBM((NS * RANGE,), jnp.int32),
    ],
)(_body)


def kernel(memory, last_update, nodes, values, ts):
    mem_out, lu_out = _dedup_gather(nodes, values, ts)
    return mem_out, lu_out
